# baseline (speedup 1.0000x reference)
import functools

import jax
import jax.numpy as jnp
from jax.experimental import pallas as pl
from jax.experimental.pallas import tpu as pltpu

K = 2048
B = 8
ROW_BLK = 256
ALPHA = 1.1


def _sor_body(x_ref, xt_ref, sel_ref, mask_ref, val_ref):
    xb = x_ref[0]
    xtb = xt_ref[0]

    c0 = xtb[0:1, :]
    c1 = xtb[1:2, :]
    c2 = xtb[2:3, :]

    n_blk = K // ROW_BLK
    for i in range(n_blk):
        r0 = x_ref[0, pl.ds(i * ROW_BLK, ROW_BLK), 0:1]
        r1 = x_ref[0, pl.ds(i * ROW_BLK, ROW_BLK), 1:2]
        r2 = x_ref[0, pl.ds(i * ROW_BLK, ROW_BLK), 2:3]

        d0 = r0 - c0
        d1 = r1 - c1
        d2c = r2 - c2
        dist = d0 * d0 + d1 * d1 + d2c * d2c

        zmask = dist == 0.0
        cnt0 = jnp.sum(zmask, axis=1, keepdims=True, dtype=jnp.int32)
        dnz = jnp.where(zmask, jnp.inf, dist)
        m2 = jnp.min(dnz, axis=1, keepdims=True)
        mmask = dnz == m2
        cnt2 = jnp.sum(mmask, axis=1, keepdims=True, dtype=jnp.int32)
        m3 = jnp.min(jnp.where(mmask, jnp.inf, dnz), axis=1, keepdims=True)

        d2 = jnp.where(cnt0 >= 2, 0.0, m2)
        d3 = jnp.where(
            cnt0 >= 3,
            0.0,
            jnp.where(
                cnt0 == 2, m2, jnp.where(cnt2 >= 2, m2, m3)
            ),
        )

        val_ref[pl.ds(i * ROW_BLK, ROW_BLK), :] = 0.5 * (d2 + d3)

    v = val_ref[:, :]
    mean = jnp.sum(v) * (1.0 / K)
    centered = v - mean
    var = jnp.sum(centered * centered) * (1.0 / (K - 1))
    threshold = mean + ALPHA * jnp.sqrt(var)
    keep = (v <= threshold).astype(jnp.float32)

    sel_ref[0] = xb * keep
    mask_ref[0] = keep


@jax.jit
def kernel(x):
    xt = jnp.swapaxes(x, 1, 2)
    _imap = lambda b: (b, b * 0, b * 0)
    sel, maskf = pl.pallas_call(
        _sor_body,
        grid=(B,),
        in_specs=[
            pl.BlockSpec((1, K, 3), _imap),
            pl.BlockSpec((1, 3, K), _imap),
        ],
        out_specs=[
            pl.BlockSpec((1, K, 3), _imap),
            pl.BlockSpec((1, K, 1), _imap),
        ],
        out_shape=[
            jax.ShapeDtypeStruct((B, K, 3), jnp.float32),
            jax.ShapeDtypeStruct((B, K, 1), jnp.float32),
        ],
        scratch_shapes=[pltpu.VMEM((K, 1), jnp.float32)],
        compiler_params=pltpu.CompilerParams(
            dimension_semantics=("parallel",),
        ),
    )(x, xt)
    mask = maskf[:, :, 0] > 0.5
    return sel, mask


# speedup vs baseline: 1.4345x; 1.4345x over previous
import functools

import jax
import jax.numpy as jnp
from jax.experimental import pallas as pl
from jax.experimental.pallas import tpu as pltpu

K = 2048
B = 8
ROW_BLK = 256
ALPHA = 1.1


def _sor_body(x_ref, xt_ref, sel_ref, mask_ref, val_ref):
    xb = x_ref[0]
    xtb = xt_ref[0]

    R = ROW_BLK
    LB = 128
    n_lb = K // LB
    inf = jnp.float32(jnp.inf)

    pat = (jax.lax.broadcasted_iota(jnp.int32, (R, LB), 1)
           - jax.lax.broadcasted_iota(jnp.int32, (R, LB), 0))

    n_blk = K // R
    for i in range(n_blk):
        r0 = x_ref[0, pl.ds(i * R, R), 0:1]
        r1 = x_ref[0, pl.ds(i * R, R), 1:2]
        r2 = x_ref[0, pl.ds(i * R, R), 2:3]

        a1 = None
        a2 = None
        for g in range(n_lb):
            c0 = xt_ref[0, 0:1, g * LB:(g + 1) * LB]
            c1 = xt_ref[0, 1:2, g * LB:(g + 1) * LB]
            c2 = xt_ref[0, 2:3, g * LB:(g + 1) * LB]
            e0 = r0 - c0
            e1 = r1 - c1
            e2 = r2 - c2
            d = e0 * e0 + e1 * e1 + e2 * e2
            off = i * R - g * LB
            if -R < off < LB:
                d = jnp.where(pat == off, inf, d)
            if g == 0:
                a1 = d
                a2 = jnp.full((R, LB), inf, dtype=jnp.float32)
            else:
                a2 = jnp.minimum(a2, jnp.maximum(a1, d))
                a1 = jnp.minimum(a1, d)

        g1 = jnp.min(a1, axis=1, keepdims=True)
        zm = a1 == g1
        cnt = jnp.sum(zm.astype(jnp.float32), axis=1, keepdims=True)
        s2 = jnp.min(jnp.where(zm, inf, a1), axis=1, keepdims=True)
        m2a = jnp.min(a2, axis=1, keepdims=True)
        d3 = jnp.where(cnt >= 2.0, g1, jnp.minimum(s2, m2a))

        val_ref[pl.ds(i * R, R), :] = 0.5 * (g1 + d3)

    v = val_ref[:, :]
    mean = jnp.sum(v) * (1.0 / K)
    centered = v - mean
    var = jnp.sum(centered * centered) * (1.0 / (K - 1))
    threshold = mean + ALPHA * jnp.sqrt(var)
    keep = (v <= threshold).astype(jnp.float32)

    sel_ref[0] = xb * keep
    mask_ref[0] = keep


@jax.jit
def kernel(x):
    xt = jnp.swapaxes(x, 1, 2)
    _imap = lambda b: (b, b * 0, b * 0)
    sel, maskf = pl.pallas_call(
        _sor_body,
        grid=(B,),
        in_specs=[
            pl.BlockSpec((1, K, 3), _imap),
            pl.BlockSpec((1, 3, K), _imap),
        ],
        out_specs=[
            pl.BlockSpec((1, K, 3), _imap),
            pl.BlockSpec((1, K, 1), _imap),
        ],
        out_shape=[
            jax.ShapeDtypeStruct((B, K, 3), jnp.float32),
            jax.ShapeDtypeStruct((B, K, 1), jnp.float32),
        ],
        scratch_shapes=[pltpu.VMEM((K, 1), jnp.float32)],
        compiler_params=pltpu.CompilerParams(
            dimension_semantics=("parallel",),
        ),
    )(x, xt)
    mask = maskf[:, :, 0] > 0.5
    return sel, mask
